# Initial kernel scaffold; baseline (speedup 1.0000x reference)
import jax
import jax.numpy as jnp
from jax.experimental import pallas as pl


def kernel(input_spikes, max_timesteps, weights, targets):
    raise NotImplementedError("write your pallas kernel here")


# speedup vs baseline: 123.6229x; 123.6229x over previous
import functools

import jax
import jax.numpy as jnp
from jax import lax
from jax.experimental import pallas as pl
from jax.experimental.pallas import tpu as pltpu
from jax.experimental.pallas import tpu_sc as plsc

N_INPUT = 1024
N_OUT = 976
N = 50000
OUT_S = N - N_OUT
FAN = 32
NT = 16
TS = 3136
NPAD = NT * TS
NVR = TS // 16
DEAD = N
CH = 16
THRESH = 0.3
NEG_INV_TAU = -1.0 / 20.0


def _snn_body(spk, mtv, wts, tgt, out_t, out_p,
              delta_sh, recv_sh,
              pot_v, lu_v, ft_v, dr_v, rc_v, zero_v, fr_v,
              w2_v, t2_v, wf_v, tf_v, one_v, spk_v, mt_v):
    wid = lax.axis_index("s")
    base = wid * TS
    iota = lax.iota(jnp.int32, 16)

    pltpu.sync_copy(mtv, mt_v)
    max_t = jnp.max(mt_v[...])
    pltpu.sync_copy(spk, spk_v)

    zf = jnp.zeros((16,), jnp.float32)

    def init_i(i, _):
        sl = pl.ds(i * 16, 16)
        pot_v[sl] = zf
        lu_v[sl] = zf
        zero_v[sl] = zf
        gi = base + i * 16 + iota
        ft_v[sl] = jnp.where(gi >= N, 0, -1)
        return 0

    lax.fori_loop(0, NVR, init_i, 0)

    def init_o(i, _):
        one_v[pl.ds(i * 16, 16)] = jnp.ones((16,), jnp.float32)
        return 0

    lax.fori_loop(0, CH * FAN // 16, init_o, 0)

    def in_i(j, c):
        i = wid * (N_INPUT // NT // 16) + j
        sl = pl.ds(i * 16, 16)
        s = spk_v[sl] > 0
        gi = i * 16 + iota
        plsc.store_compressed(fr_v.at[pl.ds(c, 16)], gi, mask=s)
        return c + jnp.max(plsc.all_reduce_population_count(s))

    c0 = lax.fori_loop(0, N_INPUT // NT // 16, in_i, jnp.int32(0))

    my_sl = pl.ds(base, TS)
    pltpu.sync_copy(zero_v, delta_sh.at[my_sl])
    pltpu.sync_copy(zero_v, recv_sh.at[my_sl])
    plsc.subcore_barrier()

    def step(t, c):
        fr_v[pl.ds(c, 16)] = jnp.full((16,), DEAD, jnp.int32)
        nch = (c + CH - 1) >> 4

        def chunk(k, _):
            idxs = fr_v.at[pl.ds(k * CH, CH)]
            pltpu.sync_copy(wts.at[idxs], w2_v)
            pltpu.sync_copy(tgt.at[idxs], t2_v)

            def flat(r, _):
                wf_v[pl.ds(r * FAN, 16)] = w2_v[r, pl.ds(0, 16)]
                wf_v[pl.ds(r * FAN + 16, 16)] = w2_v[r, pl.ds(16, 16)]
                tf_v[pl.ds(r * FAN, 16)] = t2_v[r, pl.ds(0, 16)]
                tf_v[pl.ds(r * FAN + 16, 16)] = t2_v[r, pl.ds(16, 16)]
                return 0

            lax.fori_loop(0, CH, flat, 0)
            pltpu.sync_copy(wf_v, delta_sh.at[tf_v], add=True)
            pltpu.sync_copy(one_v, recv_sh.at[tf_v], add=True)
            return 0

        lax.fori_loop(0, nch, chunk, 0)
        plsc.subcore_barrier()

        pltpu.sync_copy(delta_sh.at[my_sl], dr_v)
        pltpu.sync_copy(recv_sh.at[my_sl], rc_v)
        pltpu.sync_copy(zero_v, delta_sh.at[my_sl])
        pltpu.sync_copy(zero_v, recv_sh.at[my_sl])
        strength = jnp.where(t == 0, jnp.float32(2.0), jnp.float32(1.0))
        tf32 = t.astype(jnp.float32)

        def dense(i, cn):
            sl = pl.ds(i * 16, 16)
            m = rc_v[sl] > 0.0
            pot = pot_v[sl]
            lu = lu_v[sl]
            ft = ft_v[sl]
            dec = jnp.exp((tf32 - lu) * NEG_INV_TAU)
            pot = jnp.where(m, pot * dec + strength * dr_v[sl], pot)
            lu_v[sl] = jnp.where(m, tf32, lu)
            nf = m & (pot >= THRESH) & (ft < 0)
            pot_v[sl] = pot
            ft_v[sl] = jnp.where(nf, t, ft)
            gi = base + i * 16 + iota
            fm = nf & (gi < OUT_S)
            plsc.store_compressed(fr_v.at[pl.ds(cn, 16)], gi, mask=fm)
            return cn + jnp.max(plsc.all_reduce_population_count(fm))

        cnew = lax.fori_loop(0, NVR, dense, jnp.int32(0))
        plsc.subcore_barrier()
        return cnew

    lax.fori_loop(0, max_t, step, c0)

    @pl.when(wid == NT - 1)
    def _():
        off = OUT_S - (NT - 1) * TS
        pltpu.sync_copy(ft_v.at[pl.ds(off, N_OUT)], out_t)
        pltpu.sync_copy(pot_v.at[pl.ds(off, N_OUT)], out_p)


_snn = pl.kernel(
    _snn_body,
    out_type=[
        jax.ShapeDtypeStruct((N_OUT,), jnp.int32),
        jax.ShapeDtypeStruct((N_OUT,), jnp.float32),
    ],
    mesh=plsc.VectorSubcoreMesh(
        core_axis_name="c", subcore_axis_name="s", num_cores=1, num_subcores=NT
    ),
    compiler_params=pltpu.CompilerParams(
        needs_layout_passes=False, use_tc_tiling_on_sc=False
    ),
    scratch_types=[
        pltpu.VMEM_SHARED((NPAD,), jnp.float32),
        pltpu.VMEM_SHARED((NPAD,), jnp.float32),
        pltpu.VMEM((TS,), jnp.float32),
        pltpu.VMEM((TS,), jnp.float32),
        pltpu.VMEM((TS,), jnp.int32),
        pltpu.VMEM((TS,), jnp.float32),
        pltpu.VMEM((TS,), jnp.float32),
        pltpu.VMEM((TS,), jnp.float32),
        pltpu.VMEM((TS + 16,), jnp.int32),
        pltpu.VMEM((CH, FAN), jnp.float32),
        pltpu.VMEM((CH, FAN), jnp.int32),
        pltpu.VMEM((CH * FAN,), jnp.float32),
        pltpu.VMEM((CH * FAN,), jnp.int32),
        pltpu.VMEM((CH * FAN,), jnp.float32),
        pltpu.VMEM((N_INPUT,), jnp.int32),
        pltpu.VMEM((16,), jnp.int32),
    ],
)


def kernel(input_spikes, max_timesteps, weights, targets):
    spk = input_spikes.astype(jnp.int32)
    mt = jnp.full((16,), max_timesteps, jnp.int32)
    w = jnp.zeros((NPAD, FAN), jnp.float32).at[:N].set(weights)
    tg = jnp.full((NPAD, FAN), DEAD, jnp.int32).at[:N].set(targets)
    out_t, out_p = _snn(spk, mt, w, tg)
    return out_t, out_p
